# Initial kernel scaffold; baseline (speedup 1.0000x reference)
import jax
import jax.numpy as jnp
from jax.experimental import pallas as pl


def kernel(x, embedding):
    raise NotImplementedError("write your pallas kernel here")


# speedup vs baseline: 1.7735x; 1.7735x over previous
import functools

import jax
import jax.numpy as jnp
from jax import lax
from jax.experimental import pallas as pl
from jax.experimental.pallas import tpu as pltpu
from jax.experimental.pallas import tpu_sc as plsc

_INFO = plsc.get_sparse_core_info()
_NC, _NS = _INFO.num_cores, _INFO.num_subcores
_NW = _NC * _NS

_CH = 8
_NBUF = 2


@functools.partial(jax.jit, static_argnames=("b_per_w", "d"))
def _sc_gather(table, idx, *, b_per_w, d):
    nchunks = b_per_w // _CH
    mesh = plsc.VectorSubcoreMesh(core_axis_name="c", subcore_axis_name="s")

    @functools.partial(
        pl.kernel,
        out_type=jax.ShapeDtypeStruct((b_per_w * _NW, d), jnp.float32),
        mesh=mesh,
        scratch_types=[
            pltpu.VMEM((b_per_w,), jnp.int32),
            [pltpu.VMEM((_CH, d), jnp.float32) for _ in range(_NBUF)],
            [pltpu.SemaphoreType.DMA for _ in range(_NBUF)],
            [pltpu.SemaphoreType.DMA for _ in range(_NBUF)],
        ],
    )
    def k(table_hbm, idx_hbm, out_hbm, idx_v, bufs, gsems, osems):
        wid = lax.axis_index("s") * _NC + lax.axis_index("c")
        base = wid * b_per_w
        pltpu.sync_copy(idx_hbm.at[pl.ds(base, b_per_w)], idx_v)

        def start_gather(c, b):
            pltpu.async_copy(
                table_hbm.at[idx_v.at[pl.ds(c * _CH, _CH)]], bufs[b], gsems[b]
            )

        def drain_gather(b):
            pltpu.make_async_copy(table_hbm.at[idx_v.at[pl.ds(0, _CH)]],
                                  bufs[b], gsems[b]).wait()

        def start_out(c, b):
            pltpu.async_copy(
                bufs[b], out_hbm.at[pl.ds(base + c * _CH, _CH)], osems[b]
            )

        def drain_out(c, b):
            pltpu.make_async_copy(bufs[b],
                                  out_hbm.at[pl.ds(base + c * _CH, _CH)],
                                  osems[b]).wait()

        for b in range(_NBUF):
            start_gather(b, b)

        @pl.loop(0, nchunks - _NBUF, step=_NBUF)
        def _(c0):
            for b in range(_NBUF):
                c = c0 + b
                drain_gather(b)
                start_out(c, b)
                drain_out(c, b)
                start_gather(c + _NBUF, b)

        for b in range(_NBUF):
            c = nchunks - _NBUF + b
            drain_gather(b)
            start_out(c, b)
            drain_out(c, b)

    return k(table, idx)


def kernel(x, embedding):
    d = embedding.shape[1]
    b = x.size
    idx = x.reshape(b)
    out = _sc_gather(embedding, idx, b_per_w=b // _NW, d=d)
    return out.reshape(x.shape + (d,))
